# baseline (speedup 1.0000x reference)
import functools

import jax
import jax.numpy as jnp
from jax import lax
from jax.experimental import pallas as pl
from jax.experimental.pallas import tpu as pltpu
from jax.experimental.pallas import tpu_sc as plsc

_LANES = 16
_NSEM = 8


@functools.lru_cache(maxsize=None)
def _make_sc_gather(V, P, Dm, B, NC, NS):
    NW = NC * NS
    assert B % NW == 0
    bpw = B // NW
    assert bpw % _LANES == 0 and bpw >= _NSEM and _NSEM <= _LANES
    NG = bpw // _LANES
    assert Dm % _LANES == 0

    mesh = plsc.VectorSubcoreMesh(core_axis_name="c", subcore_axis_name="s")

    @functools.partial(
        pl.kernel,
        out_type=jax.ShapeDtypeStruct((B, P, Dm), jnp.float32),
        mesh=mesh,
        scratch_types=(
            [pltpu.VMEM((bpw,), jnp.int32)]
            + [pltpu.VMEM((1, P, Dm), jnp.float32)]
            + [pltpu.VMEM((_LANES,), jnp.int32)]
            + [pltpu.SemaphoreType.DMA] * _NSEM
        ),
    )
    def sc_fn(table_hbm, idx_hbm, flag_hbm, out_hbm, idx_v, zbuf, flag_v,
              *sems):
        wid = lax.axis_index("s") * NC + lax.axis_index("c")
        base = wid * bpw
        pltpu.sync_copy(idx_hbm.at[pl.ds(base, bpw)], idx_v)
        pltpu.sync_copy(flag_hbm, flag_v)
        member = flag_v[...][0]

        def row_copy(r, dst_row, s):
            pltpu.async_copy(
                table_hbm.at[pl.ds(r, 1)], out_hbm.at[pl.ds(dst_row, 1)],
                sems[s])

        def row_drain(s):
            pltpu.make_async_copy(
                table_hbm.at[pl.ds(0, 1)], out_hbm.at[pl.ds(base, 1)],
                sems[s]).wait()

        @pl.when(member != 0)
        def _gather_path():
            def group(j, carry):
                vec = idx_v[pl.ds(j * _LANES, _LANES)]
                for k in range(_LANES):
                    t = j * _LANES + k

                    @pl.when(t >= _NSEM)
                    def _(k=k):
                        row_drain(k % _NSEM)

                    row_copy(vec[k], base + t, k % _NSEM)
                return carry

            lax.fori_loop(0, NG, group, 0)
            for s in range(_NSEM):
                row_drain(s)

        @pl.when(member == 0)
        def _zero_path():
            zeros = jnp.zeros((_LANES,), jnp.float32)
            for p in range(P):
                def zbody(k, carry, p=p):
                    zbuf[0, p, pl.ds(k * _LANES, _LANES)] = zeros
                    return carry

                lax.fori_loop(0, Dm // _LANES, zbody, 0)

            def sbody(g, carry):
                pltpu.sync_copy(zbuf, out_hbm.at[pl.ds(base + g, 1)])
                return carry

            lax.fori_loop(0, bpw, sbody, 0)

    return sc_fn


def kernel(e_p_0, l, batch_size, task_id):
    V, P, Dm = e_p_0.shape
    B = task_id.shape[0]
    info = plsc.get_sparse_core_info()
    NC, NS = info.num_cores, info.num_subcores

    idx = task_id.astype(jnp.int32)
    is_member = jnp.any(jnp.asarray([0, 1, 2, 3, 4], jnp.int32) == l)
    flag = jnp.where(is_member, jnp.int32(1), jnp.int32(0)) + jnp.zeros(
        (_LANES,), jnp.int32)

    return _make_sc_gather(V, P, Dm, B, NC, NS)(e_p_0, idx, flag)


# speedup vs baseline: 20.7083x; 20.7083x over previous
import functools

import jax
import jax.numpy as jnp
from jax import lax
from jax.experimental import pallas as pl
from jax.experimental.pallas import tpu as pltpu
from jax.experimental.pallas import tpu_sc as plsc

_LANES = 16
_NBUF = 4


@functools.lru_cache(maxsize=None)
def _make_sc_gather(V, P, Dm, B, NC, NS):
    NW = NC * NS
    assert B % NW == 0
    bpw = B // NW
    assert bpw % _LANES == 0 and _LANES % _NBUF == 0
    NG = bpw // _LANES
    assert Dm % _LANES == 0

    mesh = plsc.VectorSubcoreMesh(core_axis_name="c", subcore_axis_name="s")

    @functools.partial(
        pl.kernel,
        out_type=jax.ShapeDtypeStruct((B, P, Dm), jnp.float32),
        mesh=mesh,
        scratch_types=(
            [pltpu.VMEM((bpw,), jnp.int32)]
            + [pltpu.VMEM((1, P, Dm), jnp.float32)] * _NBUF
            + [pltpu.VMEM((_LANES,), jnp.int32)]
            + [pltpu.SemaphoreType.DMA] * (2 * _NBUF)
        ),
    )
    def sc_fn(table_hbm, idx_hbm, flag_hbm, out_hbm, idx_v, *rest):
        bufs = rest[:_NBUF]
        flag_v = rest[_NBUF]
        gsems = rest[_NBUF + 1:2 * _NBUF + 1]
        ssems = rest[2 * _NBUF + 1:]
        wid = lax.axis_index("s") * NC + lax.axis_index("c")
        base = wid * bpw
        pltpu.sync_copy(idx_hbm.at[pl.ds(base, bpw)], idx_v)
        pltpu.sync_copy(flag_hbm, flag_v)
        member = flag_v[...][0]

        def g_start(r, b):
            pltpu.async_copy(table_hbm.at[pl.ds(r, 1)], bufs[b], gsems[b])

        def g_wait(b):
            pltpu.make_async_copy(
                table_hbm.at[pl.ds(0, 1)], bufs[b], gsems[b]).wait()

        def s_start(t, b):
            pltpu.async_copy(
                bufs[b], out_hbm.at[pl.ds(base + t, 1)], ssems[b])

        def s_drain(b):
            pltpu.make_async_copy(
                bufs[b], out_hbm.at[pl.ds(base, 1)], ssems[b]).wait()

        @pl.when(member != 0)
        def _gather_path():
            def group(j, carry):
                vec = idx_v[pl.ds(j * _LANES, _LANES)]
                for k in range(_LANES):
                    b = k % _NBUF
                    nb = (b + 2) % _NBUF
                    t = j * _LANES + k

                    @pl.when(t >= _NBUF)
                    def _(b=b):
                        s_drain(b)

                    g_start(vec[k], b)

                    @pl.when(t >= 2)
                    def _(t=t, nb=nb):
                        g_wait(nb)
                        s_start(t - 2, nb)
                return carry

            lax.fori_loop(0, NG, group, 0)
            T = bpw
            for t in (T - 2, T - 1):
                nb = t % _NBUF
                g_wait(nb)
                s_start(t, nb)
            for b in range(_NBUF):
                s_drain(b)

        @pl.when(member == 0)
        def _zero_path():
            zeros = jnp.zeros((_LANES,), jnp.float32)
            zbuf = bufs[0]
            for p in range(P):
                def zbody(k, carry, p=p):
                    zbuf[0, p, pl.ds(k * _LANES, _LANES)] = zeros
                    return carry

                lax.fori_loop(0, Dm // _LANES, zbody, 0)

            def sbody(g, carry):
                pltpu.sync_copy(zbuf, out_hbm.at[pl.ds(base + g, 1)])
                return carry

            lax.fori_loop(0, bpw, sbody, 0)

    return sc_fn


def kernel(e_p_0, l, batch_size, task_id):
    V, P, Dm = e_p_0.shape
    B = task_id.shape[0]
    info = plsc.get_sparse_core_info()
    NC, NS = info.num_cores, info.num_subcores

    idx = task_id.astype(jnp.int32)
    is_member = jnp.any(jnp.asarray([0, 1, 2, 3, 4], jnp.int32) == l)
    flag = jnp.where(is_member, jnp.int32(1), jnp.int32(0)) + jnp.zeros(
        (_LANES,), jnp.int32)

    return _make_sc_gather(V, P, Dm, B, NC, NS)(e_p_0, idx, flag)


# speedup vs baseline: 39.9710x; 1.9302x over previous
import functools

import jax
import jax.numpy as jnp
from jax import lax
from jax.experimental import pallas as pl
from jax.experimental.pallas import tpu as pltpu
from jax.experimental.pallas import tpu_sc as plsc

_LANES = 16
_NBUF = 4
_SL = 8
_LN = 128
_CH = 128


@functools.lru_cache(maxsize=None)
def _make_sc_gather(n_src, n_out, NC, NS):
    NW = NC * NS
    assert n_out % (NW * _CH) == 0
    spw = n_out // NW
    NCH = spw // _CH
    assert NCH % _NBUF == 0 and NCH >= 2 * _NBUF

    mesh = plsc.VectorSubcoreMesh(core_axis_name="c", subcore_axis_name="s")

    @functools.partial(
        pl.kernel,
        out_type=jax.ShapeDtypeStruct((n_out, _LN), jnp.float32),
        mesh=mesh,
        scratch_types=(
            [pltpu.VMEM((NCH, _CH), jnp.int32)]
            + [pltpu.VMEM((_CH, _LN), jnp.float32)] * _NBUF
            + [pltpu.VMEM((_LANES,), jnp.int32)]
            + [pltpu.SemaphoreType.DMA] * (2 * _NBUF)
        ),
    )
    def sc_fn(table_hbm, idx_hbm, flag_hbm, out_hbm, idx_v, *rest):
        bufs = rest[:_NBUF]
        flag_v = rest[_NBUF]
        gsems = rest[_NBUF + 1:2 * _NBUF + 1]
        ssems = rest[2 * _NBUF + 1:]
        wid = lax.axis_index("s") * NC + lax.axis_index("c")
        base = wid * spw
        pltpu.sync_copy(idx_hbm.at[wid], idx_v)
        pltpu.sync_copy(flag_hbm, flag_v)
        member = flag_v[...][0]

        def g_start(g, b):
            pltpu.async_copy(table_hbm.at[idx_v.at[g]], bufs[b], gsems[b])

        def g_wait(g, b):
            pltpu.make_async_copy(
                table_hbm.at[idx_v.at[g]], bufs[b], gsems[b]).wait()

        def s_start(g, b):
            pltpu.async_copy(
                bufs[b], out_hbm.at[pl.ds(base + g * _CH, _CH)], ssems[b])

        def s_drain(b):
            pltpu.make_async_copy(
                bufs[b], out_hbm.at[pl.ds(base, _CH)], ssems[b]).wait()

        @pl.when(member != 0)
        def _gather_path():
            g_start(0, 0)
            g_start(1, 1)

            def group(i, carry):
                for b in range(_NBUF):
                    g = _NBUF * i + b
                    nb = (b + 2) % _NBUF
                    g_wait(g, b)
                    s_start(g, b)

                    @pl.when(g >= 2)
                    def _(nb=nb):
                        s_drain(nb)

                    @pl.when(g + 2 < NCH)
                    def _(g=g, nb=nb):
                        g_start(g + 2, nb)
                return carry

            lax.fori_loop(0, NCH // _NBUF, group, 0)
            s_drain((NCH - 2) % _NBUF)
            s_drain((NCH - 1) % _NBUF)

        @pl.when(member == 0)
        def _zero_path():
            zeros = jnp.zeros((_LANES,), jnp.float32)
            zbuf = bufs[0]

            def zbody(k, carry):
                r = k // (_LN // _LANES)
                c = k % (_LN // _LANES)
                zbuf[r, pl.ds(c * _LANES, _LANES)] = zeros
                return carry

            lax.fori_loop(0, _CH * (_LN // _LANES), zbody, 0)

            def sbody(g, carry):
                pltpu.sync_copy(zbuf, out_hbm.at[pl.ds(base + g * _CH, _CH)])
                return carry

            lax.fori_loop(0, NCH, sbody, 0)

    return sc_fn


def kernel(e_p_0, l, batch_size, task_id):
    V, P, Dm = e_p_0.shape
    B = task_id.shape[0]
    assert V % _SL == 0 and B % _SL == 0 and Dm % _LN == 0
    NT = Dm // _LN
    info = plsc.get_sparse_core_info()
    NC, NS = info.num_cores, info.num_subcores
    NW = NC * NS

    t = e_p_0.transpose(1, 0, 2)
    t = t.reshape(P, V // _SL, _SL, NT, _LN)
    t = t.transpose(0, 1, 3, 2, 4)
    table_strips = t.reshape(P * (V // _SL) * NT * _SL, _LN)

    tid = task_id.astype(jnp.int32)
    hi = (tid // _SL).reshape(B // _SL, _SL)
    lo = (tid % _SL).reshape(B // _SL, _SL)
    p_ar = jnp.arange(P, dtype=jnp.int32)[:, None, None, None]
    ct_ar = jnp.arange(NT, dtype=jnp.int32)[None, None, :, None]
    u = ((p_ar * (V // _SL) + hi[None, :, None, :]) * NT + ct_ar) * _SL \
        + lo[None, :, None, :]
    n_out = P * (B // _SL) * NT * _SL
    idx3 = u.reshape(NW, n_out // (NW * _CH), _CH)

    is_member = jnp.any(jnp.asarray([0, 1, 2, 3, 4], jnp.int32) == l)
    flag = jnp.where(is_member, jnp.int32(1), jnp.int32(0)) + jnp.zeros(
        (_LANES,), jnp.int32)

    o = _make_sc_gather(table_strips.shape[0], n_out, NC, NS)(
        table_strips, idx3, flag)
    o = o.reshape(P, B // _SL, NT, _SL, _LN)
    o = o.transpose(0, 1, 3, 2, 4)
    o = o.reshape(P, B, Dm)
    return o.transpose(1, 0, 2)


# speedup vs baseline: 40.0362x; 1.0016x over previous
import functools

import jax
import jax.numpy as jnp
from jax import lax
from jax.experimental import pallas as pl
from jax.experimental.pallas import tpu as pltpu
from jax.experimental.pallas import tpu_sc as plsc

_LANES = 16
_NBUF = 6
_LOOK = 3
_SL = 8
_LN = 128
_CH = 128


@functools.lru_cache(maxsize=None)
def _make_sc_gather(n_src, n_out, NC, NS):
    NW = NC * NS
    assert n_out % (NW * _CH) == 0
    spw = n_out // NW
    NCH = spw // _CH
    assert NCH % _NBUF == 0 and NCH >= 2 * _NBUF

    mesh = plsc.VectorSubcoreMesh(core_axis_name="c", subcore_axis_name="s")

    @functools.partial(
        pl.kernel,
        out_type=jax.ShapeDtypeStruct((n_out, _LN), jnp.float32),
        mesh=mesh,
        scratch_types=(
            [pltpu.VMEM((NCH, _CH), jnp.int32)]
            + [pltpu.VMEM((_CH, _LN), jnp.float32)] * _NBUF
            + [pltpu.VMEM((_LANES,), jnp.int32)]
            + [pltpu.SemaphoreType.DMA] * (2 * _NBUF)
        ),
    )
    def sc_fn(table_hbm, idx_hbm, flag_hbm, out_hbm, idx_v, *rest):
        bufs = rest[:_NBUF]
        flag_v = rest[_NBUF]
        gsems = rest[_NBUF + 1:2 * _NBUF + 1]
        ssems = rest[2 * _NBUF + 1:]
        wid = lax.axis_index("s") * NC + lax.axis_index("c")
        base = wid * spw
        pltpu.sync_copy(idx_hbm.at[wid], idx_v)
        pltpu.sync_copy(flag_hbm, flag_v)
        member = flag_v[...][0]

        def g_start(g, b):
            pltpu.async_copy(table_hbm.at[idx_v.at[g]], bufs[b], gsems[b])

        def g_wait(g, b):
            pltpu.make_async_copy(
                table_hbm.at[idx_v.at[g]], bufs[b], gsems[b]).wait()

        def s_start(g, b):
            pltpu.async_copy(
                bufs[b], out_hbm.at[pl.ds(base + g * _CH, _CH)], ssems[b])

        def s_drain(b):
            pltpu.make_async_copy(
                bufs[b], out_hbm.at[pl.ds(base, _CH)], ssems[b]).wait()

        @pl.when(member != 0)
        def _gather_path():
            for g in range(_LOOK):
                g_start(g, g)

            def group(i, carry):
                for b in range(_NBUF):
                    g = _NBUF * i + b
                    nb = (b + _LOOK) % _NBUF
                    g_wait(g, b)
                    s_start(g, b)

                    @pl.when(g >= _LOOK)
                    def _(nb=nb):
                        s_drain(nb)

                    @pl.when(g + _LOOK < NCH)
                    def _(g=g, nb=nb):
                        g_start(g + _LOOK, nb)
                return carry

            lax.fori_loop(0, NCH // _NBUF, group, 0)
            for g in range(NCH - _LOOK, NCH):
                s_drain(g % _NBUF)

        @pl.when(member == 0)
        def _zero_path():
            zeros = jnp.zeros((_LANES,), jnp.float32)
            zbuf = bufs[0]

            def zbody(k, carry):
                r = k // (_LN // _LANES)
                c = k % (_LN // _LANES)
                zbuf[r, pl.ds(c * _LANES, _LANES)] = zeros
                return carry

            lax.fori_loop(0, _CH * (_LN // _LANES), zbody, 0)

            def sbody(g, carry):
                pltpu.sync_copy(zbuf, out_hbm.at[pl.ds(base + g * _CH, _CH)])
                return carry

            lax.fori_loop(0, NCH, sbody, 0)

    return sc_fn


def kernel(e_p_0, l, batch_size, task_id):
    V, P, Dm = e_p_0.shape
    B = task_id.shape[0]
    assert V % _SL == 0 and B % _SL == 0 and Dm % _LN == 0
    NT = Dm // _LN
    info = plsc.get_sparse_core_info()
    NC, NS = info.num_cores, info.num_subcores
    NW = NC * NS

    t = e_p_0.transpose(1, 0, 2)
    t = t.reshape(P, V // _SL, _SL, NT, _LN)
    t = t.transpose(0, 1, 3, 2, 4)
    table_strips = t.reshape(P * (V // _SL) * NT * _SL, _LN)

    tid = task_id.astype(jnp.int32)
    hi = (tid // _SL).reshape(B // _SL, _SL)
    lo = (tid % _SL).reshape(B // _SL, _SL)
    p_ar = jnp.arange(P, dtype=jnp.int32)[:, None, None, None]
    ct_ar = jnp.arange(NT, dtype=jnp.int32)[None, None, :, None]
    u = ((p_ar * (V // _SL) + hi[None, :, None, :]) * NT + ct_ar) * _SL \
        + lo[None, :, None, :]
    n_out = P * (B // _SL) * NT * _SL
    idx3 = u.reshape(NW, n_out // (NW * _CH), _CH)

    is_member = jnp.any(jnp.asarray([0, 1, 2, 3, 4], jnp.int32) == l)
    flag = jnp.where(is_member, jnp.int32(1), jnp.int32(0)) + jnp.zeros(
        (_LANES,), jnp.int32)

    o = _make_sc_gather(table_strips.shape[0], n_out, NC, NS)(
        table_strips, idx3, flag)
    o = o.reshape(P, B // _SL, NT, _SL, _LN)
    o = o.transpose(0, 1, 3, 2, 4)
    o = o.reshape(P, B, Dm)
    return o.transpose(1, 0, 2)


# speedup vs baseline: 47.5684x; 1.1881x over previous
import functools

import jax
import jax.numpy as jnp
from jax import lax
from jax.experimental import pallas as pl
from jax.experimental.pallas import tpu as pltpu
from jax.experimental.pallas import tpu_sc as plsc

_LANES = 16
_NBUF = 6
_LOOK = 3
_SL = 8
_LN = 128
_CH = 128


@functools.lru_cache(maxsize=None)
def _make_sc_gather(V, P, NT, B, NC, NS):
    NW = NC * NS
    n_out = P * B * NT
    assert n_out % (NW * _CH) == 0
    spw = n_out // NW
    NCH = spw // _CH
    assert NCH % _NBUF == 0 and NCH >= 2 * _NBUF
    PLANE = (B // _SL) * NT * _SL
    assert (B // _SL) & (B // _SL - 1) == 0 and NT == 6
    assert P * B * NT // _SL < 786432
    BT_BITS = (B // _SL).bit_length() - 1

    mesh = plsc.VectorSubcoreMesh(core_axis_name="c", subcore_axis_name="s")

    @functools.partial(
        pl.kernel,
        out_type=jax.ShapeDtypeStruct((n_out, _LN), jnp.float32),
        mesh=mesh,
        compiler_params=pltpu.CompilerParams(needs_layout_passes=False),
        scratch_types=(
            [pltpu.VMEM((B,), jnp.int32)]
            + [pltpu.VMEM((NCH, _CH), jnp.int32)]
            + [pltpu.VMEM((_CH, _LN), jnp.float32)] * _NBUF
            + [pltpu.VMEM((_LANES,), jnp.int32)]
            + [pltpu.SemaphoreType.DMA] * (2 * _NBUF)
        ),
    )
    def sc_fn(table_hbm, tid_hbm, flag_hbm, out_hbm, tid_v, idx_v, *rest):
        bufs = rest[:_NBUF]
        flag_v = rest[_NBUF]
        gsems = rest[_NBUF + 1:2 * _NBUF + 1]
        ssems = rest[2 * _NBUF + 1:]
        wid = lax.axis_index("s") * NC + lax.axis_index("c")
        base = wid * spw
        pltpu.sync_copy(tid_hbm, tid_v)
        pltpu.sync_copy(flag_hbm, flag_v)
        member = flag_v[...][0]
        lanes = lax.broadcasted_iota(jnp.int32, (_LANES,), 0)

        def compute_idx(g):
            for i in range(_CH // _LANES):
                U = base + g * _CH + i * _LANES + lanes
                s = U & (_SL - 1)
                Q = U >> 3
                t6 = ((Q.astype(jnp.uint32) * jnp.uint32(43691))
                      >> jnp.uint32(18)).astype(jnp.int32)
                ct = Q - t6 * NT
                p = t6 >> BT_BITS
                bt = t6 & (B // _SL - 1)
                r = plsc.load_gather(tid_v, [bt * _SL + s])
                u = ((p * (V // _SL) + (r >> 3)) * NT + ct) * _SL + (r & 7)
                idx_v[g, pl.ds(i * _LANES, _LANES)] = u

        def g_start(g, b):
            pltpu.async_copy(table_hbm.at[idx_v.at[g]], bufs[b], gsems[b])

        def g_wait(g, b):
            pltpu.make_async_copy(
                table_hbm.at[idx_v.at[g]], bufs[b], gsems[b]).wait()

        def s_start(g, b):
            pltpu.async_copy(
                bufs[b], out_hbm.at[pl.ds(base + g * _CH, _CH)], ssems[b])

        def s_drain(b):
            pltpu.make_async_copy(
                bufs[b], out_hbm.at[pl.ds(base, _CH)], ssems[b]).wait()

        @pl.when(member != 0)
        def _gather_path():
            for g in range(_LOOK):
                compute_idx(g)
                g_start(g, g)

            def group(i, carry):
                for b in range(_NBUF):
                    g = _NBUF * i + b
                    nb = (b + _LOOK) % _NBUF
                    g_wait(g, b)
                    s_start(g, b)

                    @pl.when(g >= _LOOK)
                    def _(nb=nb):
                        s_drain(nb)

                    @pl.when(g + _LOOK < NCH)
                    def _(g=g, nb=nb):
                        compute_idx(g + _LOOK)
                        g_start(g + _LOOK, nb)
                return carry

            lax.fori_loop(0, NCH // _NBUF, group, 0)
            for g in range(NCH - _LOOK, NCH):
                s_drain(g % _NBUF)

        @pl.when(member == 0)
        def _zero_path():
            zeros = jnp.zeros((_LANES,), jnp.float32)
            zbuf = bufs[0]

            def zbody(k, carry):
                r = k // (_LN // _LANES)
                c = k % (_LN // _LANES)
                zbuf[r, pl.ds(c * _LANES, _LANES)] = zeros
                return carry

            lax.fori_loop(0, _CH * (_LN // _LANES), zbody, 0)

            def sbody(g, carry):
                pltpu.sync_copy(zbuf, out_hbm.at[pl.ds(base + g * _CH, _CH)])
                return carry

            lax.fori_loop(0, NCH, sbody, 0)

    return sc_fn


def kernel(e_p_0, l, batch_size, task_id):
    V, P, Dm = e_p_0.shape
    B = task_id.shape[0]
    assert V % _SL == 0 and B % _SL == 0 and Dm % _LN == 0
    NT = Dm // _LN
    info = plsc.get_sparse_core_info()
    NC, NS = info.num_cores, info.num_subcores

    t = e_p_0.transpose(1, 0, 2)
    t = t.reshape(P, V // _SL, _SL, NT, _LN)
    t = t.transpose(0, 1, 3, 2, 4)
    table_strips = t.reshape(P * V * NT, _LN)

    is_member = jnp.any(jnp.asarray([0, 1, 2, 3, 4], jnp.int32) == l)
    flag = jnp.where(is_member, jnp.int32(1), jnp.int32(0)) + jnp.zeros(
        (_LANES,), jnp.int32)

    o = _make_sc_gather(V, P, NT, B, NC, NS)(
        table_strips, task_id.astype(jnp.int32), flag)
    o = o.reshape(P, B // _SL, NT, _SL, _LN)
    o = o.transpose(0, 1, 3, 2, 4)
    o = o.reshape(P, B, Dm)
    return o.transpose(1, 0, 2)


# speedup vs baseline: 47.5850x; 1.0003x over previous
import functools

import jax
import jax.numpy as jnp
from jax import lax
from jax.experimental import pallas as pl
from jax.experimental.pallas import tpu as pltpu
from jax.experimental.pallas import tpu_sc as plsc

_LANES = 16
_NBUF = 6
_LOOK = 4
_SL = 8
_LN = 128
_CH = 128


@functools.lru_cache(maxsize=None)
def _make_sc_gather(V, P, NT, B, NC, NS):
    NW = NC * NS
    n_out = P * B * NT
    assert n_out % (NW * _CH) == 0
    spw = n_out // NW
    NCH = spw // _CH
    assert NCH % _NBUF == 0 and NCH >= 2 * _NBUF
    PLANE = (B // _SL) * NT * _SL
    assert (B // _SL) & (B // _SL - 1) == 0 and NT == 6
    assert P * B * NT // _SL < 786432
    BT_BITS = (B // _SL).bit_length() - 1

    mesh = plsc.VectorSubcoreMesh(core_axis_name="c", subcore_axis_name="s")

    @functools.partial(
        pl.kernel,
        out_type=jax.ShapeDtypeStruct((n_out, _LN), jnp.float32),
        mesh=mesh,
        compiler_params=pltpu.CompilerParams(needs_layout_passes=False),
        scratch_types=(
            [pltpu.VMEM((B,), jnp.int32)]
            + [pltpu.VMEM((NCH, _CH), jnp.int32)]
            + [pltpu.VMEM((_CH, _LN), jnp.float32)] * _NBUF
            + [pltpu.VMEM((_LANES,), jnp.int32)]
            + [pltpu.SemaphoreType.DMA] * (2 * _NBUF)
        ),
    )
    def sc_fn(table_hbm, tid_hbm, flag_hbm, out_hbm, tid_v, idx_v, *rest):
        bufs = rest[:_NBUF]
        flag_v = rest[_NBUF]
        gsems = rest[_NBUF + 1:2 * _NBUF + 1]
        ssems = rest[2 * _NBUF + 1:]
        wid = lax.axis_index("s") * NC + lax.axis_index("c")
        base = wid * spw
        pltpu.sync_copy(tid_hbm, tid_v)
        pltpu.sync_copy(flag_hbm, flag_v)
        member = flag_v[...][0]
        lanes = lax.broadcasted_iota(jnp.int32, (_LANES,), 0)

        def compute_idx(g):
            for i in range(_CH // _LANES):
                U = base + g * _CH + i * _LANES + lanes
                s = U & (_SL - 1)
                Q = U >> 3
                t6 = ((Q.astype(jnp.uint32) * jnp.uint32(43691))
                      >> jnp.uint32(18)).astype(jnp.int32)
                ct = Q - t6 * NT
                p = t6 >> BT_BITS
                bt = t6 & (B // _SL - 1)
                r = plsc.load_gather(tid_v, [bt * _SL + s])
                u = ((p * (V // _SL) + (r >> 3)) * NT + ct) * _SL + (r & 7)
                idx_v[g, pl.ds(i * _LANES, _LANES)] = u

        def g_start(g, b):
            pltpu.async_copy(table_hbm.at[idx_v.at[g]], bufs[b], gsems[b])

        def g_wait(g, b):
            pltpu.make_async_copy(
                table_hbm.at[idx_v.at[g]], bufs[b], gsems[b]).wait()

        def s_start(g, b):
            pltpu.async_copy(
                bufs[b], out_hbm.at[pl.ds(base + g * _CH, _CH)], ssems[b])

        def s_drain(b):
            pltpu.make_async_copy(
                bufs[b], out_hbm.at[pl.ds(base, _CH)], ssems[b]).wait()

        @pl.when(member != 0)
        def _gather_path():
            for g in range(_LOOK):
                compute_idx(g)
                g_start(g, g)

            def group(i, carry):
                for b in range(_NBUF):
                    g = _NBUF * i + b
                    nb = (b + _LOOK) % _NBUF
                    g_wait(g, b)
                    s_start(g, b)

                    @pl.when(g >= _NBUF - _LOOK)
                    def _(nb=nb):
                        s_drain(nb)

                    @pl.when(g + _LOOK < NCH)
                    def _(g=g, nb=nb):
                        compute_idx(g + _LOOK)
                        g_start(g + _LOOK, nb)
                return carry

            lax.fori_loop(0, NCH // _NBUF, group, 0)
            for g in range(NCH - (_NBUF - _LOOK), NCH):
                s_drain(g % _NBUF)

        @pl.when(member == 0)
        def _zero_path():
            zeros = jnp.zeros((_LANES,), jnp.float32)
            zbuf = bufs[0]

            def zbody(k, carry):
                r = k // (_LN // _LANES)
                c = k % (_LN // _LANES)
                zbuf[r, pl.ds(c * _LANES, _LANES)] = zeros
                return carry

            lax.fori_loop(0, _CH * (_LN // _LANES), zbody, 0)

            def sbody(g, carry):
                pltpu.sync_copy(zbuf, out_hbm.at[pl.ds(base + g * _CH, _CH)])
                return carry

            lax.fori_loop(0, NCH, sbody, 0)

    return sc_fn


def kernel(e_p_0, l, batch_size, task_id):
    V, P, Dm = e_p_0.shape
    B = task_id.shape[0]
    assert V % _SL == 0 and B % _SL == 0 and Dm % _LN == 0
    NT = Dm // _LN
    info = plsc.get_sparse_core_info()
    NC, NS = info.num_cores, info.num_subcores

    t = e_p_0.transpose(1, 0, 2)
    t = t.reshape(P, V // _SL, _SL, NT, _LN)
    t = t.transpose(0, 1, 3, 2, 4)
    table_strips = t.reshape(P * V * NT, _LN)

    is_member = jnp.any(jnp.asarray([0, 1, 2, 3, 4], jnp.int32) == l)
    flag = jnp.where(is_member, jnp.int32(1), jnp.int32(0)) + jnp.zeros(
        (_LANES,), jnp.int32)

    o = _make_sc_gather(V, P, NT, B, NC, NS)(
        table_strips, task_id.astype(jnp.int32), flag)
    o = o.reshape(P, B // _SL, NT, _SL, _LN)
    o = o.transpose(0, 1, 3, 2, 4)
    o = o.reshape(P, B, Dm)
    return o.transpose(1, 0, 2)
